# baseline (speedup 1.0000x reference)
import jax
import jax.numpy as jnp
from jax.experimental import pallas as pl
from jax.experimental.pallas import tpu as pltpu

_BIT = 8
_N_LEVELS = 2.0 ** _BIT - 1.0


def _body(in_ref, out_ref, buf_ref, m_ref):
    p = pl.program_id(0)
    i = pl.program_id(1)
    bc = in_ref.shape[1]

    @pl.when(p == 0)
    def _phase_load():
        x = in_ref[...]
        buf_ref[:, pl.ds(i * bc, bc)] = x
        m = jnp.max(jnp.abs(x))

        @pl.when(i == 0)
        def _init():
            m_ref[0] = m

        @pl.when(i > 0)
        def _acc():
            m_ref[0] = jnp.maximum(m_ref[0], m)

    @pl.when(p == 1)
    def _phase_quant():
        alpha = m_ref[0]
        scale = _N_LEVELS / jnp.maximum(2.0 * alpha, 1e-8)
        zero_point = scale * (-alpha)
        x = buf_ref[:, pl.ds(i * bc, bc)]
        q = jnp.round(scale * x - zero_point)
        q = jnp.clip(q, 0.0, _N_LEVELS)
        out_ref[...] = (q + zero_point) / scale


def kernel(tensor, image_size):
    rows, cols = tensor.shape
    nb = 16
    bc = cols // nb

    in_spec = pl.BlockSpec(
        (rows, bc), lambda p, i: (0, jnp.where(p == 0, i, 0)))
    out_spec = pl.BlockSpec(
        (rows, bc), lambda p, i: (0, jnp.where(p == 0, 0, i)))

    return pl.pallas_call(
        _body,
        grid=(2, nb),
        in_specs=[in_spec],
        out_specs=out_spec,
        out_shape=jax.ShapeDtypeStruct((rows, cols), tensor.dtype),
        scratch_shapes=[
            pltpu.VMEM((rows, cols), jnp.float32),
            pltpu.SMEM((1,), jnp.float32),
        ],
    )(tensor)


# speedup vs baseline: 1.3267x; 1.3267x over previous
import jax
import jax.numpy as jnp
from jax.experimental import pallas as pl
from jax.experimental.pallas import tpu as pltpu

_BIT = 8
_N_LEVELS = 2.0 ** _BIT - 1.0


def _body(in_ref, out_ref, buf_ref, m_ref):
    p = pl.program_id(0)
    i = pl.program_id(1)
    bc = in_ref.shape[1]

    @pl.when(p == 0)
    def _phase_load():
        x = in_ref[...]
        buf_ref[:, pl.ds(i * bc, bc)] = x
        m = jnp.max(jnp.abs(x))

        @pl.when(i == 0)
        def _init():
            m_ref[0] = m

        @pl.when(i > 0)
        def _acc():
            m_ref[0] = jnp.maximum(m_ref[0], m)

    @pl.when(p == 1)
    def _phase_quant():
        alpha = m_ref[0]
        scale = _N_LEVELS / jnp.maximum(2.0 * alpha, 1e-8)
        zero_point = scale * (-alpha)
        x = buf_ref[:, pl.ds(i * bc, bc)]
        q = jnp.round(scale * x - zero_point)
        q = jnp.clip(q, 0.0, _N_LEVELS)
        out_ref[...] = (q + zero_point) / scale


def kernel(tensor, image_size):
    rows, cols = tensor.shape
    nb = 8
    bc = cols // nb

    in_spec = pl.BlockSpec(
        (rows, bc), lambda p, i: (0, jnp.where(p == 0, i, 0)))
    out_spec = pl.BlockSpec(
        (rows, bc), lambda p, i: (0, jnp.where(p == 0, 0, i)))

    return pl.pallas_call(
        _body,
        grid=(2, nb),
        in_specs=[in_spec],
        out_specs=out_spec,
        out_shape=jax.ShapeDtypeStruct((rows, cols), tensor.dtype),
        scratch_shapes=[
            pltpu.VMEM((rows, cols), jnp.float32),
            pltpu.SMEM((1,), jnp.float32),
        ],
    )(tensor)


# speedup vs baseline: 1.4899x; 1.1230x over previous
import jax
import jax.numpy as jnp
from jax.experimental import pallas as pl
from jax.experimental.pallas import tpu as pltpu

_BIT = 8
_N_LEVELS = 2.0 ** _BIT - 1.0


def _body(in_ref, out_ref, buf_ref, m_ref):
    p = pl.program_id(0)
    i = pl.program_id(1)
    bc = in_ref.shape[1]

    @pl.when(p == 0)
    def _phase_load():
        x = in_ref[...]
        buf_ref[:, pl.ds(i * bc, bc)] = x
        m = jnp.max(jnp.abs(x))

        @pl.when(i == 0)
        def _init():
            m_ref[0] = m

        @pl.when(i > 0)
        def _acc():
            m_ref[0] = jnp.maximum(m_ref[0], m)

    @pl.when(p == 1)
    def _phase_quant():
        alpha = m_ref[0]
        scale = _N_LEVELS / jnp.maximum(2.0 * alpha, 1e-8)
        zero_point = scale * (-alpha)
        x = buf_ref[:, pl.ds(i * bc, bc)]
        q = jnp.round(scale * x - zero_point)
        q = jnp.clip(q, 0.0, _N_LEVELS)
        out_ref[...] = (q + zero_point) / scale


def kernel(tensor, image_size):
    rows, cols = tensor.shape
    nb = 4
    bc = cols // nb

    in_spec = pl.BlockSpec(
        (rows, bc), lambda p, i: (0, jnp.where(p == 0, i, 0)))
    out_spec = pl.BlockSpec(
        (rows, bc), lambda p, i: (0, jnp.where(p == 0, 0, i)))

    return pl.pallas_call(
        _body,
        grid=(2, nb),
        in_specs=[in_spec],
        out_specs=out_spec,
        out_shape=jax.ShapeDtypeStruct((rows, cols), tensor.dtype),
        scratch_shapes=[
            pltpu.VMEM((rows, cols), jnp.float32),
            pltpu.SMEM((1,), jnp.float32),
        ],
    )(tensor)
